# Initial kernel scaffold; baseline (speedup 1.0000x reference)
import jax
import jax.numpy as jnp
from jax.experimental import pallas as pl


def kernel(batch, table):
    raise NotImplementedError("write your pallas kernel here")


# speedup vs baseline: 14.1496x; 14.1496x over previous
import functools

import jax
import jax.numpy as jnp
from jax import lax
from jax.experimental import pallas as pl
from jax.experimental.pallas import tpu as pltpu
from jax.experimental.pallas import tpu_sc as plsc

_IW = 128
_NW = 32


def _gather_kernel(n_rows, sub):
  chunk = sub * _IW
  per_w = n_rows // _NW
  n_chunks = per_w // chunk
  assert per_w % chunk == 0
  ne = chunk * 10
  mesh = plsc.VectorSubcoreMesh(core_axis_name="c", subcore_axis_name="s")

  @functools.partial(
      pl.kernel,
      mesh=mesh,
      out_type=jax.ShapeDtypeStruct((n_rows * 10,), jnp.float32),
      scratch_types=[
          pltpu.VMEM((chunk,), jnp.int32),
          pltpu.VMEM((chunk,), jnp.int32),
          pltpu.VMEM((chunk,), jnp.int32),
          pltpu.VMEM((chunk,), jnp.int32),
          pltpu.VMEM((2 * chunk, 8), jnp.float32),
          pltpu.VMEM((ne,), jnp.float32),
          pltpu.SemaphoreType.DMA,
      ],
      compiler_params=pltpu.CompilerParams(
          use_tc_tiling_on_sc=False, needs_layout_passes=False),
  )
  def k(idx_hbm, tab8_hbm, out_hbm, idx_v, q0_v, q1_v, off_v, rows_v,
        comp_v, sem):
    nc = lax.axis_size("c")
    wid = lax.axis_index("s") * nc + lax.axis_index("c")
    base = wid * per_w

    def body(c, carry):
      row0 = base + c * chunk
      pltpu.sync_copy(idx_hbm.at[pl.ds(row0, chunk)], idx_v)

      def vecs(u, carry):
        t = u * 16
        i = idx_v[pl.ds(t, 16)]
        q0 = i + (i >> 2)
        q0_v[pl.ds(t, 16)] = q0
        q1_v[pl.ds(t, 16)] = q0 + 1
        off_v[pl.ds(t, 16)] = (i & 3) * 2
        return carry
      lax.fori_loop(0, chunk // 16, vecs, 0)

      for j in range(sub):
        pltpu.async_copy(tab8_hbm.at[q0_v.at[pl.ds(j * _IW, _IW)]],
                         rows_v.at[pl.ds(j * _IW, _IW)], sem)
        pltpu.async_copy(tab8_hbm.at[q1_v.at[pl.ds(j * _IW, _IW)]],
                         rows_v.at[pl.ds(chunk + j * _IW, _IW)], sem)
      for j in range(sub):
        pltpu.make_async_copy(tab8_hbm.at[q0_v.at[pl.ds(j * _IW, _IW)]],
                              rows_v.at[pl.ds(j * _IW, _IW)], sem).wait()
        pltpu.make_async_copy(tab8_hbm.at[q1_v.at[pl.ds(j * _IW, _IW)]],
                              rows_v.at[pl.ds(chunk + j * _IW, _IW)], sem).wait()

      def comp(g, carry):
        for p in range(5):
          eb = g * 80 + p * 16
          e = lax.iota(jnp.int32, 16) + eb
          m = (e * 52429) >> 19
          t = plsc.load_gather(off_v, [m]) + (e - m * 10)
          vals = plsc.load_gather(rows_v, [m + (t >> 3) * chunk, t & 7])
          comp_v[pl.ds(eb, 16)] = vals
        return carry
      lax.fori_loop(0, ne // 80, comp, 0)

      pltpu.sync_copy(comp_v, out_hbm.at[pl.ds(row0 * 10, ne)])
      return carry

    lax.fori_loop(0, n_chunks, body, 0)

  return k


def kernel(batch, table):
  b, l = batch.shape
  vocab, embed = table.shape
  n = b * l
  flat_idx = batch.reshape(n).astype(jnp.int32)
  tab8 = table.reshape(vocab * embed // 8, 8)
  out = _gather_kernel(n, 8)(flat_idx, tab8)
  return out.reshape(b, l, embed)


# speedup vs baseline: 39.4906x; 2.7909x over previous
import functools

import jax
import jax.numpy as jnp
from jax import lax
from jax.experimental import pallas as pl
from jax.experimental.pallas import tpu as pltpu
from jax.experimental.pallas import tpu_sc as plsc

_NW = 32
_BB = 128
_LC = 8


def _gather_kernel(nb, nl, ne):
  per_w = nb // _NW
  nblk = per_w // _BB
  nlc = nl // _LC
  cn = _BB * _LC
  mesh = plsc.VectorSubcoreMesh(core_axis_name="c", subcore_axis_name="s")

  @functools.partial(
      pl.kernel,
      mesh=mesh,
      out_type=jax.ShapeDtypeStruct((ne, nl, nb), jnp.float32),
      scratch_types=[
          pltpu.VMEM((_BB, nl), jnp.int32),
          pltpu.VMEM((cn,), jnp.int32),
          pltpu.VMEM((cn,), jnp.int32),
          pltpu.VMEM((cn,), jnp.int32),
          pltpu.VMEM((2 * cn, 8), jnp.float32),
          pltpu.VMEM((ne, _LC, _BB), jnp.float32),
          pltpu.SemaphoreType.DMA,
          pltpu.SemaphoreType.DMA,
      ],
      compiler_params=pltpu.CompilerParams(
          use_tc_tiling_on_sc=False, needs_layout_passes=False),
  )
  def k(idx_hbm, tab8_hbm, out_hbm, idx_v, q0_v, q1_v, off_v, rows_v,
        comp_v, gsem, osem):
    nc = lax.axis_size("c")
    wid = lax.axis_index("s") * nc + lax.axis_index("c")
    b_base = wid * per_w
    iota = lax.iota(jnp.int32, 16)

    def blk(ib, carry):
      b0 = b_base + ib * _BB
      pltpu.sync_copy(idx_hbm.at[pl.ds(b0, _BB)], idx_v)

      def unit(il, carry):
        l0 = il * _LC

        def vecs(u, carry):
          m = u * 16 + iota
          i = plsc.load_gather(idx_v, [m >> 3, l0 + (m & 7)])
          q0 = i + (i >> 2)
          q0_v[pl.ds(u * 16, 16)] = q0
          q1_v[pl.ds(u * 16, 16)] = q0 + 1
          off_v[pl.ds(u * 16, 16)] = (i & 3) * 2
          return carry
        lax.fori_loop(0, cn // 16, vecs, 0)

        for j in range(cn // _BB):
          pltpu.async_copy(tab8_hbm.at[q0_v.at[pl.ds(j * _BB, _BB)]],
                           rows_v.at[pl.ds(j * _BB, _BB)], gsem)
          pltpu.async_copy(tab8_hbm.at[q1_v.at[pl.ds(j * _BB, _BB)]],
                           rows_v.at[pl.ds(cn + j * _BB, _BB)], gsem)
        for j in range(cn // _BB):
          pltpu.make_async_copy(tab8_hbm.at[q0_v.at[pl.ds(j * _BB, _BB)]],
                                rows_v.at[pl.ds(j * _BB, _BB)], gsem).wait()
          pltpu.make_async_copy(tab8_hbm.at[q1_v.at[pl.ds(j * _BB, _BB)]],
                                rows_v.at[pl.ds(cn + j * _BB, _BB)], gsem).wait()

        def comp(z, carry):
          l_in = z >> 3
          b_sub = z & 7
          m_vec = (b_sub * 16 + iota) * _LC + l_in
          off16 = plsc.load_gather(off_v, [m_vec])
          for e in range(ne):
            t = off16 + e
            vals = plsc.load_gather(rows_v, [m_vec + (t >> 3) * cn, t & 7])
            comp_v[e, l_in, pl.ds(b_sub * 16, 16)] = vals
          return carry
        lax.fori_loop(0, _LC * _BB // 16, comp, 0)

        pltpu.async_copy(
            comp_v,
            out_hbm.at[:, pl.ds(l0, _LC), pl.ds(b0, _BB)], osem)
        pltpu.make_async_copy(
            comp_v,
            out_hbm.at[:, pl.ds(l0, _LC), pl.ds(b0, _BB)], osem).wait()
        return carry

      lax.fori_loop(0, nlc, unit, 0)
      return carry

    lax.fori_loop(0, nblk, blk, 0)

  return k


def kernel(batch, table):
  b, l = batch.shape
  vocab, embed = table.shape
  tab8 = table.reshape(vocab * embed // 8, 8)
  out_t = _gather_kernel(b, l, embed)(batch.astype(jnp.int32), tab8)
  return jnp.transpose(out_t, (2, 1, 0))
